# baseline (speedup 1.0000x reference)
import functools

import numpy as np

import jax
import jax.numpy as jnp
from jax import lax
from jax.experimental import pallas as pl
from jax.experimental.pallas import tpu as pltpu
from jax.experimental.pallas import tpu_sc as plsc

_N = 10000
_D = 128
_E = 320000

_NC = 2
_NS = 16
_NW = _NC * _NS

_CH = 128
_TOT = _E // _CH
_TOTP = 2560
_BLK = 32
_KLO = 78
_K = 80

_NACC = 10112
_RPT = _NACC // _NS
_NHIST = 10240
_HPT = _NHIST // _NS

_BR = 1000


def _mesh():
    return plsc.VectorSubcoreMesh(core_axis_name="c", subcore_axis_name="s")


def _sc_degree(dstw):
    @functools.partial(
        pl.kernel,
        out_type=jax.ShapeDtypeStruct((_NC, _NHIST), jnp.float32),
        mesh=_mesh(),
        scratch_types=[
            pltpu.VMEM((_K + 8, _CH), jnp.int32),
            pltpu.VMEM((_CH,), jnp.float32),
            pltpu.VMEM((_HPT,), jnp.float32),
            pltpu.VMEM_SHARED((_NHIST,), jnp.float32),
        ],
    )
    def kdeg(dst_hbm, out_hbm, dst_v, ones_v, zero_v, hist_sh):
        c = lax.axis_index("c")
        s = lax.axis_index("s")
        wid = c * _NS + s
        kc = jnp.where(wid < 2, _K, _KLO)
        base = _KLO * wid + 2 * jnp.minimum(wid, 2)
        abase = (base // 8) * 8
        off = base - abase
        pltpu.sync_copy(dst_hbm.at[pl.ds(abase, _K + 8)], dst_v)
        for t in range(_CH // 16):
            ones_v[pl.ds(t * 16, 16)] = jnp.full((16,), 1.0, jnp.float32)
        for t in range(_HPT // 16):
            zero_v[pl.ds(t * 16, 16)] = jnp.zeros((16,), jnp.float32)
        pltpu.sync_copy(zero_v, hist_sh.at[pl.ds(s * _HPT, _HPT)])
        plsc.subcore_barrier()

        def body(j, carry):
            pltpu.sync_copy(ones_v, hist_sh.at[dst_v.at[off + j]], add=True)
            return carry

        lax.fori_loop(0, kc, body, 0)
        plsc.subcore_barrier()
        pltpu.sync_copy(hist_sh.at[pl.ds(s * _HPT, _HPT)],
                        out_hbm.at[c, pl.ds(s * _HPT, _HPT)])

    return kdeg(dstw)


def _sc_scatter(g, srcw, dstw, zrows):
    @functools.partial(
        pl.kernel,
        out_type=jax.ShapeDtypeStruct((_NC, _NACC, _D), jnp.float32),
        mesh=_mesh(),
        scratch_types=[
            pltpu.VMEM((_BLK + 8, _CH), jnp.int32),
            pltpu.VMEM((_BLK + 8, _CH), jnp.int32),
            pltpu.VMEM((_BLK + 8, _CH), jnp.int32),
            pltpu.VMEM((_CH, _D), jnp.float32),
            pltpu.VMEM((_CH, _D), jnp.float32),
            pltpu.VMEM_SHARED((_NACC, _D), jnp.float32),
            pltpu.SemaphoreType.DMA,
            pltpu.SemaphoreType.DMA,
        ],
    )
    def kconv(g_hbm, src_hbm, dst_hbm, z_hbm, out_hbm,
              srca_v, srcb_v, dst_v, bufa, bufb, acc_sh, sema, semb):
        c = lax.axis_index("c")
        s = lax.axis_index("s")
        wid = c * _NS + s
        kc = jnp.where(wid < 2, _K, _KLO)
        base = _KLO * wid + 2 * jnp.minimum(wid, 2)
        abase = (base // 8) * 8
        off = base - abase
        pltpu.sync_copy(src_hbm.at[pl.ds(abase, _BLK + 8)], srca_v)
        pltpu.sync_copy(dst_hbm.at[pl.ds(abase, _BLK + 8)], dst_v)

        def loc(j):
            return off + lax.rem(j, _BLK)

        def gather(jl, src_v, buf, sem):
            pltpu.async_copy(g_hbm.at[src_v.at[jl]], buf, sem)

        def gather_p(j, buf, sem):
            p = lax.rem(j // _BLK, 2)

            @pl.when(p == 0)
            def _():
                gather(loc(j), srca_v, buf, sem)

            @pl.when(p == 1)
            def _():
                gather(loc(j), srcb_v, buf, sem)

        def gwait(buf, sem):
            pltpu.make_async_copy(g_hbm.at[srca_v.at[0]], buf, sem).wait()

        def scat(jl, buf):
            pltpu.sync_copy(buf, acc_sh.at[dst_v.at[jl]], add=True)

        gather(loc(0), srca_v, bufa, sema)
        gather(loc(1), srca_v, bufb, semb)
        pltpu.sync_copy(z_hbm.at[pl.ds(s * _RPT, _RPT)],
                        acc_sh.at[pl.ds(s * _RPT, _RPT)])
        plsc.subcore_barrier()
        nb2 = _BLK // 2

        def body(jj, carry):
            j = 2 * jj

            @pl.when(jnp.logical_and(lax.rem(jj, nb2) == 0, jj > 0))
            def _():
                pltpu.sync_copy(
                    dst_hbm.at[pl.ds(abase + (jj // nb2) * _BLK, _BLK + 8)],
                    dst_v)

            gwait(bufa, sema)
            scat(loc(j), bufa)

            @pl.when(lax.rem(jj, nb2) == nb2 - 1)
            def _():
                nxt = jj // nb2 + 1
                pn = lax.rem(nxt, 2)

                @pl.when(pn == 0)
                def _():
                    pltpu.sync_copy(
                        src_hbm.at[pl.ds(abase + nxt * _BLK, _BLK + 8)],
                        srca_v)

                @pl.when(pn == 1)
                def _():
                    pltpu.sync_copy(
                        src_hbm.at[pl.ds(abase + nxt * _BLK, _BLK + 8)],
                        srcb_v)

            gather_p(j + 2, bufa, sema)
            gwait(bufb, semb)
            scat(loc(j + 1), bufb)
            gather_p(j + 3, bufb, semb)
            return carry

        lax.fori_loop(0, kc // 2 - 1, body, 0)
        gwait(bufa, sema)
        scat(loc(kc - 2), bufa)
        gwait(bufb, semb)
        scat(loc(kc - 1), bufb)
        plsc.subcore_barrier()
        pltpu.sync_copy(acc_sh.at[pl.ds(s * _RPT, _RPT)],
                        out_hbm.at[c, pl.ds(s * _RPT, _RPT)])

    return kconv(g, srcw, dstw, zrows)


def _tc_first(h0, h1, x, W1):
    def body(h0_ref, h1_ref, x_ref, w_ref, g_ref, dinv_ref):
        deg = h0_ref[...] + h1_ref[...] + 1.0
        dinv = lax.rsqrt(deg)
        dinv_ref[...] = dinv
        g_ref[...] = dinv * jnp.dot(x_ref[...], w_ref[...],
                                    preferred_element_type=jnp.float32, precision=lax.Precision.HIGHEST)

    return pl.pallas_call(
        body,
        grid=(_N // _BR,),
        in_specs=[
            pl.BlockSpec((_BR, 1), lambda i: (i, 0)),
            pl.BlockSpec((_BR, 1), lambda i: (i, 0)),
            pl.BlockSpec((_BR, _D), lambda i: (i, 0)),
            pl.BlockSpec((_D, _D), lambda i: (0, 0)),
        ],
        out_specs=[
            pl.BlockSpec((_BR, _D), lambda i: (i, 0)),
            pl.BlockSpec((_BR, 1), lambda i: (i, 0)),
        ],
        out_shape=[
            jax.ShapeDtypeStruct((_N, _D), jnp.float32),
            jax.ShapeDtypeStruct((_N, 1), jnp.float32),
        ],
    )(h0, h1, x, W1)


def _tc_mid(acc, g, dinv, b, W):
    def body(a0_ref, a1_ref, g_ref, dinv_ref, b_ref, w_ref, out_ref):
        dinv = dinv_ref[...]
        h = jnp.maximum(
            dinv * (a0_ref[0] + a1_ref[0] + g_ref[...]) + b_ref[...], 0.0)
        out_ref[...] = dinv * jnp.dot(h, w_ref[...],
                                      preferred_element_type=jnp.float32, precision=lax.Precision.HIGHEST)

    return pl.pallas_call(
        body,
        grid=(_N // _BR,),
        in_specs=[
            pl.BlockSpec((1, _BR, _D), lambda i: (0, i, 0)),
            pl.BlockSpec((1, _BR, _D), lambda i: (1, i, 0)),
            pl.BlockSpec((_BR, _D), lambda i: (i, 0)),
            pl.BlockSpec((_BR, 1), lambda i: (i, 0)),
            pl.BlockSpec((1, _D), lambda i: (0, 0)),
            pl.BlockSpec((_D, _D), lambda i: (0, 0)),
        ],
        out_specs=pl.BlockSpec((_BR, _D), lambda i: (i, 0)),
        out_shape=jax.ShapeDtypeStruct((_N, _D), jnp.float32),
    )(acc, acc, g, dinv, b, W)


def _tc_last(acc, g, dinv, b3, Wm1, bm1, Wm2, bm2):
    def body(a0_ref, a1_ref, g_ref, dinv_ref, b3_ref, wm1_ref, bm1_ref,
             wm2_ref, bm2_ref, out_ref):
        h3 = (dinv_ref[...] * (a0_ref[0] + a1_ref[0] + g_ref[...])
              + b3_ref[...])
        m = jnp.maximum(
            jnp.dot(h3, wm1_ref[...], preferred_element_type=jnp.float32, precision=lax.Precision.HIGHEST)
            + bm1_ref[...], 0.0)
        out_ref[...] = (jnp.dot(m, wm2_ref[...],
                                preferred_element_type=jnp.float32, precision=lax.Precision.HIGHEST)
                        + bm2_ref[...])

    return pl.pallas_call(
        body,
        grid=(_N // _BR,),
        in_specs=[
            pl.BlockSpec((1, _BR, _D), lambda i: (0, i, 0)),
            pl.BlockSpec((1, _BR, _D), lambda i: (1, i, 0)),
            pl.BlockSpec((_BR, _D), lambda i: (i, 0)),
            pl.BlockSpec((_BR, 1), lambda i: (i, 0)),
            pl.BlockSpec((1, _D), lambda i: (0, 0)),
            pl.BlockSpec((_D, _D), lambda i: (0, 0)),
            pl.BlockSpec((1, _D), lambda i: (0, 0)),
            pl.BlockSpec((_D, 1), lambda i: (0, 0)),
            pl.BlockSpec((1, 1), lambda i: (0, 0)),
        ],
        out_specs=pl.BlockSpec((_BR, 1), lambda i: (i, 0)),
        out_shape=jax.ShapeDtypeStruct((_N, 1), jnp.float32),
    )(acc, acc, g, dinv, b3, Wm1, bm1, Wm2, bm2)


def kernel(x, edge_index, W1, b1, W2, b2, W3, b3, Wm1, bm1, Wm2, bm2):
    src = edge_index[0]
    dst = edge_index[1]
    pad = (_TOTP - _TOT) * _CH
    srcw = jnp.concatenate(
        [src, jnp.zeros((pad,), jnp.int32)]).reshape(_TOTP, _CH)
    dstw = jnp.concatenate(
        [dst, jnp.zeros((pad,), jnp.int32)]).reshape(_TOTP, _CH)
    zrows = jnp.zeros((_NACC, _D), jnp.float32)

    hist = _sc_degree(dstw)
    h0 = hist[0, :_N].reshape(_N, 1)
    h1 = hist[1, :_N].reshape(_N, 1)

    g1, dinv = _tc_first(h0, h1, x, W1)
    acc1 = _sc_scatter(g1, srcw, dstw, zrows)
    g2 = _tc_mid(acc1, g1, dinv, b1.reshape(1, _D), W2)
    acc2 = _sc_scatter(g2, srcw, dstw, zrows)
    g3 = _tc_mid(acc2, g2, dinv, b2.reshape(1, _D), W3)
    acc3 = _sc_scatter(g3, srcw, dstw, zrows)
    out = _tc_last(acc3, g3, dinv, b3.reshape(1, _D), Wm1,
                   bm1.reshape(1, _D), Wm2, bm2.reshape(1, 1))
    return out


# speedup vs baseline: 1.0599x; 1.0599x over previous
import functools

import numpy as np

import jax
import jax.numpy as jnp
from jax import lax
from jax.experimental import pallas as pl
from jax.experimental.pallas import tpu as pltpu
from jax.experimental.pallas import tpu_sc as plsc

_N = 10000
_D = 128
_E = 320000

_NC = 2
_NS = 16
_NW = _NC * _NS

_CH = 128
_TOT = _E // _CH
_TOTP = 2560
_BLK = 32
_KLO = 78
_K = 80

_NACC = 10112
_RPT = _NACC // _NS
_NHIST = 10240
_HPT = _NHIST // _NS

_BR = 2000


def _mesh():
    return plsc.VectorSubcoreMesh(core_axis_name="c", subcore_axis_name="s")


def _sc_degree(dstw):
    @functools.partial(
        pl.kernel,
        out_type=jax.ShapeDtypeStruct((_NC, _NHIST), jnp.float32),
        mesh=_mesh(),
        scratch_types=[
            pltpu.VMEM((_K + 8, _CH), jnp.int32),
            pltpu.VMEM((_CH,), jnp.float32),
            pltpu.VMEM((_HPT,), jnp.float32),
            pltpu.VMEM_SHARED((_NHIST,), jnp.float32),
        ],
    )
    def kdeg(dst_hbm, out_hbm, dst_v, ones_v, zero_v, hist_sh):
        c = lax.axis_index("c")
        s = lax.axis_index("s")
        wid = c * _NS + s
        kc = jnp.where(wid < 2, _K, _KLO)
        base = _KLO * wid + 2 * jnp.minimum(wid, 2)
        abase = (base // 8) * 8
        off = base - abase
        pltpu.sync_copy(dst_hbm.at[pl.ds(abase, _K + 8)], dst_v)
        for t in range(_CH // 16):
            ones_v[pl.ds(t * 16, 16)] = jnp.full((16,), 1.0, jnp.float32)
        for t in range(_HPT // 16):
            zero_v[pl.ds(t * 16, 16)] = jnp.zeros((16,), jnp.float32)
        pltpu.sync_copy(zero_v, hist_sh.at[pl.ds(s * _HPT, _HPT)])
        plsc.subcore_barrier()

        def body(j, carry):
            pltpu.sync_copy(ones_v, hist_sh.at[dst_v.at[off + j]], add=True)
            return carry

        lax.fori_loop(0, kc, body, 0)
        plsc.subcore_barrier()
        pltpu.sync_copy(hist_sh.at[pl.ds(s * _HPT, _HPT)],
                        out_hbm.at[c, pl.ds(s * _HPT, _HPT)])

    return kdeg(dstw)


def _sc_scatter(g, srcw, dstw, zrows):
    @functools.partial(
        pl.kernel,
        out_type=jax.ShapeDtypeStruct((_NC, _NACC, _D), jnp.float32),
        mesh=_mesh(),
        scratch_types=[
            pltpu.VMEM((_BLK + 8, _CH), jnp.int32),
            pltpu.VMEM((_BLK + 8, _CH), jnp.int32),
            pltpu.VMEM((_BLK + 8, _CH), jnp.int32),
            pltpu.VMEM((_CH, _D), jnp.float32),
            pltpu.VMEM((_CH, _D), jnp.float32),
            pltpu.VMEM_SHARED((_NACC, _D), jnp.float32),
            pltpu.SemaphoreType.DMA,
            pltpu.SemaphoreType.DMA,
        ],
    )
    def kconv(g_hbm, src_hbm, dst_hbm, z_hbm, out_hbm,
              srca_v, srcb_v, dst_v, bufa, bufb, acc_sh, sema, semb):
        c = lax.axis_index("c")
        s = lax.axis_index("s")
        wid = c * _NS + s
        kc = jnp.where(wid < 2, _K, _KLO)
        base = _KLO * wid + 2 * jnp.minimum(wid, 2)
        abase = (base // 8) * 8
        off = base - abase
        pltpu.sync_copy(src_hbm.at[pl.ds(abase, _BLK + 8)], srca_v)
        pltpu.sync_copy(dst_hbm.at[pl.ds(abase, _BLK + 8)], dst_v)

        def loc(j):
            return off + lax.rem(j, _BLK)

        def gather(jl, src_v, buf, sem):
            pltpu.async_copy(g_hbm.at[src_v.at[jl]], buf, sem)

        def gather_p(j, buf, sem):
            p = lax.rem(j // _BLK, 2)

            @pl.when(p == 0)
            def _():
                gather(loc(j), srca_v, buf, sem)

            @pl.when(p == 1)
            def _():
                gather(loc(j), srcb_v, buf, sem)

        def gwait(buf, sem):
            pltpu.make_async_copy(g_hbm.at[srca_v.at[0]], buf, sem).wait()

        def scat(jl, buf):
            pltpu.sync_copy(buf, acc_sh.at[dst_v.at[jl]], add=True)

        gather(loc(0), srca_v, bufa, sema)
        gather(loc(1), srca_v, bufb, semb)
        pltpu.sync_copy(z_hbm.at[pl.ds(s * _RPT, _RPT)],
                        acc_sh.at[pl.ds(s * _RPT, _RPT)])
        plsc.subcore_barrier()
        nb2 = _BLK // 2

        def body(jj, carry):
            j = 2 * jj

            @pl.when(jnp.logical_and(lax.rem(jj, nb2) == 0, jj > 0))
            def _():
                pltpu.sync_copy(
                    dst_hbm.at[pl.ds(abase + (jj // nb2) * _BLK, _BLK + 8)],
                    dst_v)

            gwait(bufa, sema)
            scat(loc(j), bufa)

            @pl.when(lax.rem(jj, nb2) == nb2 - 1)
            def _():
                nxt = jj // nb2 + 1
                pn = lax.rem(nxt, 2)

                @pl.when(pn == 0)
                def _():
                    pltpu.sync_copy(
                        src_hbm.at[pl.ds(abase + nxt * _BLK, _BLK + 8)],
                        srca_v)

                @pl.when(pn == 1)
                def _():
                    pltpu.sync_copy(
                        src_hbm.at[pl.ds(abase + nxt * _BLK, _BLK + 8)],
                        srcb_v)

            gather_p(j + 2, bufa, sema)
            gwait(bufb, semb)
            scat(loc(j + 1), bufb)
            gather_p(j + 3, bufb, semb)
            return carry

        lax.fori_loop(0, kc // 2 - 1, body, 0)
        gwait(bufa, sema)
        scat(loc(kc - 2), bufa)
        gwait(bufb, semb)
        scat(loc(kc - 1), bufb)
        plsc.subcore_barrier()
        pltpu.sync_copy(acc_sh.at[pl.ds(s * _RPT, _RPT)],
                        out_hbm.at[c, pl.ds(s * _RPT, _RPT)])

    return kconv(g, srcw, dstw, zrows)


def _tc_first(h0, h1, x, W1):
    def body(h0_ref, h1_ref, x_ref, w_ref, g_ref, dinv_ref):
        deg = h0_ref[...] + h1_ref[...] + 1.0
        dinv = lax.rsqrt(deg)
        dinv_ref[...] = dinv
        g_ref[...] = dinv * jnp.dot(x_ref[...], w_ref[...],
                                    preferred_element_type=jnp.float32, precision=lax.Precision.HIGHEST)

    return pl.pallas_call(
        body,
        grid=(_N // _BR,),
        in_specs=[
            pl.BlockSpec((_BR, 1), lambda i: (i, 0)),
            pl.BlockSpec((_BR, 1), lambda i: (i, 0)),
            pl.BlockSpec((_BR, _D), lambda i: (i, 0)),
            pl.BlockSpec((_D, _D), lambda i: (0, 0)),
        ],
        out_specs=[
            pl.BlockSpec((_BR, _D), lambda i: (i, 0)),
            pl.BlockSpec((_BR, 1), lambda i: (i, 0)),
        ],
        out_shape=[
            jax.ShapeDtypeStruct((_N, _D), jnp.float32),
            jax.ShapeDtypeStruct((_N, 1), jnp.float32),
        ],
    )(h0, h1, x, W1)


def _tc_mid(acc, g, dinv, b, W):
    def body(a0_ref, a1_ref, g_ref, dinv_ref, b_ref, w_ref, out_ref):
        dinv = dinv_ref[...]
        h = jnp.maximum(
            dinv * (a0_ref[0] + a1_ref[0] + g_ref[...]) + b_ref[...], 0.0)
        out_ref[...] = dinv * jnp.dot(h, w_ref[...],
                                      preferred_element_type=jnp.float32, precision=lax.Precision.HIGHEST)

    return pl.pallas_call(
        body,
        grid=(_N // _BR,),
        in_specs=[
            pl.BlockSpec((1, _BR, _D), lambda i: (0, i, 0)),
            pl.BlockSpec((1, _BR, _D), lambda i: (1, i, 0)),
            pl.BlockSpec((_BR, _D), lambda i: (i, 0)),
            pl.BlockSpec((_BR, 1), lambda i: (i, 0)),
            pl.BlockSpec((1, _D), lambda i: (0, 0)),
            pl.BlockSpec((_D, _D), lambda i: (0, 0)),
        ],
        out_specs=pl.BlockSpec((_BR, _D), lambda i: (i, 0)),
        out_shape=jax.ShapeDtypeStruct((_N, _D), jnp.float32),
    )(acc, acc, g, dinv, b, W)


def _tc_last(acc, g, dinv, b3, Wm1, bm1, Wm2, bm2):
    def body(a0_ref, a1_ref, g_ref, dinv_ref, b3_ref, wm1_ref, bm1_ref,
             wm2_ref, bm2_ref, out_ref):
        h3 = (dinv_ref[...] * (a0_ref[0] + a1_ref[0] + g_ref[...])
              + b3_ref[...])
        m = jnp.maximum(
            jnp.dot(h3, wm1_ref[...], preferred_element_type=jnp.float32, precision=lax.Precision.HIGHEST)
            + bm1_ref[...], 0.0)
        out_ref[...] = (jnp.dot(m, wm2_ref[...],
                                preferred_element_type=jnp.float32, precision=lax.Precision.HIGHEST)
                        + bm2_ref[...])

    return pl.pallas_call(
        body,
        grid=(_N // _BR,),
        in_specs=[
            pl.BlockSpec((1, _BR, _D), lambda i: (0, i, 0)),
            pl.BlockSpec((1, _BR, _D), lambda i: (1, i, 0)),
            pl.BlockSpec((_BR, _D), lambda i: (i, 0)),
            pl.BlockSpec((_BR, 1), lambda i: (i, 0)),
            pl.BlockSpec((1, _D), lambda i: (0, 0)),
            pl.BlockSpec((_D, _D), lambda i: (0, 0)),
            pl.BlockSpec((1, _D), lambda i: (0, 0)),
            pl.BlockSpec((_D, 1), lambda i: (0, 0)),
            pl.BlockSpec((1, 1), lambda i: (0, 0)),
        ],
        out_specs=pl.BlockSpec((_BR, 1), lambda i: (i, 0)),
        out_shape=jax.ShapeDtypeStruct((_N, 1), jnp.float32),
    )(acc, acc, g, dinv, b3, Wm1, bm1, Wm2, bm2)


def kernel(x, edge_index, W1, b1, W2, b2, W3, b3, Wm1, bm1, Wm2, bm2):
    src = edge_index[0]
    dst = edge_index[1]
    pad = (_TOTP - _TOT) * _CH
    srcw = jnp.concatenate(
        [src, jnp.zeros((pad,), jnp.int32)]).reshape(_TOTP, _CH)
    dstw = jnp.concatenate(
        [dst, jnp.zeros((pad,), jnp.int32)]).reshape(_TOTP, _CH)
    zrows = jnp.zeros((_NACC, _D), jnp.float32)

    hist = _sc_degree(dstw)
    h0 = hist[0, :_N].reshape(_N, 1)
    h1 = hist[1, :_N].reshape(_N, 1)

    g1, dinv = _tc_first(h0, h1, x, W1)
    acc1 = _sc_scatter(g1, srcw, dstw, zrows)
    g2 = _tc_mid(acc1, g1, dinv, b1.reshape(1, _D), W2)
    acc2 = _sc_scatter(g2, srcw, dstw, zrows)
    g3 = _tc_mid(acc2, g2, dinv, b2.reshape(1, _D), W3)
    acc3 = _sc_scatter(g3, srcw, dstw, zrows)
    out = _tc_last(acc3, g3, dinv, b3.reshape(1, _D), Wm1,
                   bm1.reshape(1, _D), Wm2, bm2.reshape(1, 1))
    return out
